# baseline (speedup 1.0000x reference)
import functools

import jax
import jax.numpy as jnp
from jax import lax
from jax.experimental import pallas as pl
from jax.experimental.pallas import tpu as pltpu


def _ceil_to(x, m):
    return (x + m - 1) // m * m


def _conv_pool_stats_kernel(x_ref, w_ref, y_ref, pool_ref, stats_ref, *,
                            c_in, ch3, ho, wo, wop, w2,
                            mask_bot, mask_right, n_per):
    for n in range(n_per):
        _one_image(x_ref, w_ref, y_ref, pool_ref, stats_ref, n,
                   c_in=c_in, ch3=ch3, ho=ho, wo=wo, wop=wop, w2=w2,
                   mask_bot=mask_bot, mask_right=mask_right)


def _one_image(x_ref, w_ref, y_ref, pool_ref, stats_ref, n, *,
               c_in, ch3, ho, wo, wop, w2, mask_bot, mask_right):
    xb = x_ref[n].astype(jnp.bfloat16)
    mrows = ho * wop
    c2 = 2 * c_in
    xe = xb[:, 0][0:ho]
    xo = xb[:, 1][0:ho]
    padr = wop - w2

    p1 = jnp.pad(xe, ((0, 0), (1, padr), (0, 0)))
    p2 = jnp.pad(xo, ((0, 0), (1, padr), (0, 0)))
    p0 = jnp.pad(xo[0:ho - 1], ((1, 0), (1, padr), (0, 0)))

    y = jnp.zeros((mrows, ch3), jnp.float32)
    for kh, p in ((0, p0), (1, p1), (2, p2)):
        main = p[:, 1:1 + wop, :].reshape(mrows, c2)
        shif = p[:, 0:wop, :].reshape(mrows, c2)
        y = y + jnp.dot(main, w_ref[kh * 2 * c2:kh * 2 * c2 + c2],
                        preferred_element_type=jnp.float32)
        y = y + jnp.dot(shif, w_ref[kh * 2 * c2 + c2:(kh + 1) * 2 * c2],
                        preferred_element_type=jnp.float32)
    y_ref[n] = y.astype(jnp.bfloat16)

    owp_idx = lax.rem(lax.broadcasted_iota(jnp.int32, (mrows, 1), 0),
                      jnp.int32(wop))
    ys = jnp.where(owp_idx < wo, y, 0.0)
    stats_ref[n, 0:1, :] = jnp.sum(ys, axis=0, keepdims=True)
    stats_ref[n, 1:2, :] = jnp.sum(ys * ys, axis=0, keepdims=True)

    neg = jnp.bfloat16(-1e30)
    shp = (ho, wop + 1, c2)
    d0 = lax.broadcasted_iota(jnp.int32, shp, 0)
    p0m = jnp.where(d0 == 0, neg, p0)
    p2m = jnp.where(d0 == ho - 1, neg, p2) if mask_bot else p2
    m1 = jnp.maximum(jnp.maximum(p0m, p1), p2m)
    d1 = lax.broadcasted_iota(jnp.int32, shp, 1)
    dl = lax.broadcasted_iota(jnp.int32, shp, 2)
    m1 = jnp.where((d1 == 0) & (dl >= c_in), neg, m1)
    if mask_right:
        m1 = jnp.where((d1 == wo) & (dl >= c_in), neg, m1)
    pa = m1[:, 1:1 + wop, :].reshape(mrows, c2)
    pb = m1[:, 0:wop, :].reshape(mrows, c2)
    pool = jnp.maximum(jnp.maximum(pa[:, 0:c_in], pa[:, c_in:c2]),
                       pb[:, c_in:c2])
    pool_ref[n] = pool


def _apply_kernel(y_ref, pool_ref, stats_ref, out_ref, *, ch3, c_in, m_total, eps):
    ssum = jnp.sum(stats_ref[:, 0, :], axis=0, keepdims=True)
    ssq = jnp.sum(stats_ref[:, 1, :], axis=0, keepdims=True)
    mean = ssum / m_total
    var = jnp.maximum(ssq / m_total - mean * mean, 0.0)
    inv_std = lax.rsqrt(var + eps)
    shift = -mean * inv_std
    y = y_ref[...].astype(jnp.float32)
    out_ref[:, 0:ch3] = jnp.maximum(y * inv_std + shift, 0.0)
    out_ref[:, ch3:ch3 + c_in] = pool_ref[...].astype(jnp.float32)


def kernel(x_nchw, w_oihw, bias, *, eps=1e-5):
    del bias

    N, C_in, H, W = x_nchw.shape
    Ch3 = w_oihw.shape[0]
    Ho = (H + 2 - 3) // 2 + 1
    Wo = (W + 2 - 3) // 2 + 1
    Wop = _ceil_to(Wo, 8)
    M = N * Ho * Wo
    Mr = Ho * Wop
    Cout = Ch3 + C_in

    cparams = pltpu.CompilerParams(dimension_semantics=("parallel",),
                                   vmem_limit_bytes=48 * 1024 * 1024)
    x = jnp.transpose(x_nchw, (0, 2, 3, 1))
    He, We = H + H % 2, W + W % 2
    if He != H or We != W:
        x = jnp.pad(x, ((0, 0), (0, He - H), (0, We - W), (0, 0)))
    xpp = x.reshape(N, He // 2, 2, We // 2, 2 * C_in)
    W2 = We // 2

    wt = jnp.transpose(w_oihw, (2, 3, 1, 0)).astype(jnp.bfloat16)
    blocks = []
    for kh in range(3):
        blocks.append(wt[kh, 1])
        blocks.append(wt[kh, 2])
        blocks.append(jnp.zeros((C_in, Ch3), jnp.bfloat16))
        blocks.append(wt[kh, 0])
    wcat = jnp.concatenate(blocks, axis=0)

    mask_bot = 2 * Ho - 1 >= H
    mask_right = 2 * Wo - 1 >= W
    NP = 4 if N % 4 == 0 else (2 if N % 2 == 0 else 1)

    cost1 = pl.CostEstimate(
        flops=2 * N * Mr * 6 * 2 * C_in * Ch3 + 12 * N * Mr * C_in,
        transcendentals=0,
        bytes_accessed=2 * N * (He * W2 * 2 * C_in + Mr * (Ch3 + C_in)) + 4 * N * 8 * Ch3)
    y_bf, pool_bf, stats = pl.pallas_call(
        functools.partial(_conv_pool_stats_kernel, c_in=C_in, ch3=Ch3,
                          ho=Ho, wo=Wo, wop=Wop, w2=W2,
                          mask_bot=mask_bot, mask_right=mask_right, n_per=NP),
        out_shape=(jax.ShapeDtypeStruct((N, Mr, Ch3), jnp.bfloat16),
                   jax.ShapeDtypeStruct((N, Mr, C_in), jnp.bfloat16),
                   jax.ShapeDtypeStruct((N, 8, Ch3), jnp.float32)),
        grid=(N // NP,),
        in_specs=[pl.BlockSpec((NP, He // 2, 2, W2, 2 * C_in),
                               lambda i: (i, 0, 0, 0, 0)),
                  pl.BlockSpec((12 * C_in, Ch3), lambda i: (0, 0))],
        out_specs=(pl.BlockSpec((NP, Mr, Ch3), lambda i: (i, 0, 0)),
                   pl.BlockSpec((NP, Mr, C_in), lambda i: (i, 0, 0)),
                   pl.BlockSpec((NP, 8, Ch3), lambda i: (i, 0, 0))),
        compiler_params=cparams,
        cost_estimate=cost1,
    )(xpp, wcat)

    mean = jnp.sum(stats[:, 0, :], axis=0) / M
    var = jnp.maximum(jnp.sum(stats[:, 1, :], axis=0) / M - mean * mean, 0.0)
    inv_std = lax.rsqrt(var + eps)
    conv = jnp.maximum(y_bf.astype(jnp.float32) * inv_std - mean * inv_std, 0.0)
    fused = jnp.concatenate([conv, pool_bf.astype(jnp.float32)], axis=2)
    out = fused.reshape(N, Ho, Wop, Cout)[:, :, :Wo, :]
    return jnp.transpose(out, (0, 3, 1, 2))


# speedup vs baseline: 1.1539x; 1.1539x over previous
import functools

import jax
import jax.numpy as jnp
from jax import lax
from jax.experimental import pallas as pl
from jax.experimental.pallas import tpu as pltpu


def _ceil_to(x, m):
    return (x + m - 1) // m * m


def _conv_pool_stats_kernel(x_ref, w_ref, y_ref, pool_ref, stats_ref, *,
                            c_in, ch3, ho, wo, wop, w2,
                            mask_bot, mask_right, n_per):
    for n in range(n_per):
        _one_image(x_ref, w_ref, y_ref, pool_ref, stats_ref, n,
                   c_in=c_in, ch3=ch3, ho=ho, wo=wo, wop=wop, w2=w2,
                   mask_bot=mask_bot, mask_right=mask_right)


def _one_image(x_ref, w_ref, y_ref, pool_ref, stats_ref, n, *,
               c_in, ch3, ho, wo, wop, w2, mask_bot, mask_right):
    xb = x_ref[n].astype(jnp.bfloat16)
    mrows = ho * wop
    c2 = 2 * c_in
    xe = xb[:, 0][0:ho]
    xo = xb[:, 1][0:ho]
    padr = wop - w2

    p1 = jnp.pad(xe, ((0, 0), (1, padr), (0, 0)))
    p2 = jnp.pad(xo, ((0, 0), (1, padr), (0, 0)))
    p0 = jnp.pad(xo[0:ho - 1], ((1, 0), (1, padr), (0, 0)))

    y = jnp.zeros((mrows, ch3), jnp.float32)
    for kh, p in ((0, p0), (1, p1), (2, p2)):
        main = p[:, 1:1 + wop, :].reshape(mrows, c2)
        shif = p[:, 0:wop, :].reshape(mrows, c2)
        y = y + jnp.dot(main, w_ref[kh * 2 * c2:kh * 2 * c2 + c2],
                        preferred_element_type=jnp.float32)
        y = y + jnp.dot(shif, w_ref[kh * 2 * c2 + c2:(kh + 1) * 2 * c2],
                        preferred_element_type=jnp.float32)
    y_ref[n] = y.astype(jnp.bfloat16)

    owp_idx = lax.rem(lax.broadcasted_iota(jnp.int32, (mrows, 1), 0),
                      jnp.int32(wop))
    ys = jnp.where(owp_idx < wo, y, 0.0)
    stats_ref[n, 0:1, :] = jnp.sum(ys, axis=0, keepdims=True)
    stats_ref[n, 1:2, :] = jnp.sum(ys * ys, axis=0, keepdims=True)

    neg = jnp.bfloat16(-1e30)
    shp = (ho, wop + 1, c2)
    d0 = lax.broadcasted_iota(jnp.int32, shp, 0)
    p0m = jnp.where(d0 == 0, neg, p0)
    p2m = jnp.where(d0 == ho - 1, neg, p2) if mask_bot else p2
    m1 = jnp.maximum(jnp.maximum(p0m, p1), p2m)
    d1 = lax.broadcasted_iota(jnp.int32, shp, 1)
    dl = lax.broadcasted_iota(jnp.int32, shp, 2)
    m1 = jnp.where((d1 == 0) & (dl >= c_in), neg, m1)
    if mask_right:
        m1 = jnp.where((d1 == wo) & (dl >= c_in), neg, m1)
    pa = m1[:, 1:1 + wop, :].reshape(mrows, c2)
    pb = m1[:, 0:wop, :].reshape(mrows, c2)
    pool = jnp.maximum(jnp.maximum(pa[:, 0:c_in], pa[:, c_in:c2]),
                       pb[:, c_in:c2])
    pool_ref[n] = pool


def _apply_kernel(y_ref, pool_ref, stats_ref, out_ref, *, ch3, c_in, m_total, eps):
    ssum = jnp.sum(stats_ref[:, 0, :], axis=0, keepdims=True)
    ssq = jnp.sum(stats_ref[:, 1, :], axis=0, keepdims=True)
    mean = ssum / m_total
    var = jnp.maximum(ssq / m_total - mean * mean, 0.0)
    inv_std = lax.rsqrt(var + eps)
    shift = -mean * inv_std
    y = y_ref[...].astype(jnp.float32)
    out_ref[:, 0:ch3] = jnp.maximum(y * inv_std + shift, 0.0)
    out_ref[:, ch3:ch3 + c_in] = pool_ref[...].astype(jnp.float32)


def kernel(x_nchw, w_oihw, bias, *, eps=1e-5):
    del bias

    N, C_in, H, W = x_nchw.shape
    Ch3 = w_oihw.shape[0]
    Ho = (H + 2 - 3) // 2 + 1
    Wo = (W + 2 - 3) // 2 + 1
    Wop = _ceil_to(Wo, 8)
    M = N * Ho * Wo
    Mr = Ho * Wop
    Cout = Ch3 + C_in

    cparams = pltpu.CompilerParams(dimension_semantics=("parallel",),
                                   vmem_limit_bytes=48 * 1024 * 1024)
    x = jnp.transpose(x_nchw, (0, 2, 3, 1))
    He, We = H + H % 2, W + W % 2
    if He != H or We != W:
        x = jnp.pad(x, ((0, 0), (0, He - H), (0, We - W), (0, 0)))
    xpp = x.reshape(N, He // 2, 2, We // 2, 2 * C_in)
    W2 = We // 2

    wt = jnp.transpose(w_oihw, (2, 3, 1, 0)).astype(jnp.bfloat16)
    blocks = []
    for kh in range(3):
        blocks.append(wt[kh, 1])
        blocks.append(wt[kh, 2])
        blocks.append(jnp.zeros((C_in, Ch3), jnp.bfloat16))
        blocks.append(wt[kh, 0])
    wcat = jnp.concatenate(blocks, axis=0)

    mask_bot = 2 * Ho - 1 >= H
    mask_right = 2 * Wo - 1 >= W
    NP = 4 if N % 4 == 0 else (2 if N % 2 == 0 else 1)

    cost1 = pl.CostEstimate(
        flops=2 * N * Mr * 6 * 2 * C_in * Ch3 + 12 * N * Mr * C_in,
        transcendentals=0,
        bytes_accessed=2 * N * (He * W2 * 2 * C_in + Mr * (Ch3 + C_in)) + 4 * N * 8 * Ch3)
    y_bf, pool_bf, stats = pl.pallas_call(
        functools.partial(_conv_pool_stats_kernel, c_in=C_in, ch3=Ch3,
                          ho=Ho, wo=Wo, wop=Wop, w2=W2,
                          mask_bot=mask_bot, mask_right=mask_right, n_per=NP),
        out_shape=(jax.ShapeDtypeStruct((N, Mr, Ch3), jnp.bfloat16),
                   jax.ShapeDtypeStruct((N, Mr, C_in), jnp.bfloat16),
                   jax.ShapeDtypeStruct((N, 8, Ch3), jnp.float32)),
        grid=(N // NP,),
        in_specs=[pl.BlockSpec((NP, He // 2, 2, W2, 2 * C_in),
                               lambda i: (i, 0, 0, 0, 0)),
                  pl.BlockSpec((12 * C_in, Ch3), lambda i: (0, 0))],
        out_specs=(pl.BlockSpec((NP, Mr, Ch3), lambda i: (i, 0, 0)),
                   pl.BlockSpec((NP, Mr, C_in), lambda i: (i, 0, 0)),
                   pl.BlockSpec((NP, 8, Ch3), lambda i: (i, 0, 0))),
        compiler_params=cparams,
        cost_estimate=cost1,
    )(xpp, wcat)

    M2 = N * Mr
    TM = min(2048, M2)
    n2 = _ceil_to(M2, TM) // TM
    cost2 = pl.CostEstimate(
        flops=3 * M2 * Ch3,
        transcendentals=0,
        bytes_accessed=2 * M2 * (Ch3 + C_in) + 4 * M2 * (Ch3 + C_in))
    fused = pl.pallas_call(
        functools.partial(_apply_kernel, ch3=Ch3, c_in=C_in, m_total=M, eps=eps),
        out_shape=jax.ShapeDtypeStruct((M2, Cout), jnp.float32),
        grid=(n2,),
        in_specs=[pl.BlockSpec((TM, Ch3), lambda i: (i, 0)),
                  pl.BlockSpec((TM, C_in), lambda i: (i, 0)),
                  pl.BlockSpec((N, 8, Ch3), lambda i: (0, 0, 0))],
        out_specs=pl.BlockSpec((TM, Cout), lambda i: (i, 0)),
        compiler_params=cparams,
        cost_estimate=cost2,
    )(y_bf.reshape(M2, Ch3), pool_bf.reshape(M2, C_in), stats)

    out = fused.reshape(N, Ho, Wop, Cout)[:, :, :Wo, :]
    return jnp.transpose(out, (0, 3, 1, 2))


# speedup vs baseline: 1.1906x; 1.0317x over previous
import functools

import jax
import jax.numpy as jnp
from jax import lax
from jax.experimental import pallas as pl
from jax.experimental.pallas import tpu as pltpu


def _ceil_to(x, m):
    return (x + m - 1) // m * m


def _conv_pool_stats_kernel(x_ref, w_ref, y_ref, pool_ref, stats_ref, *,
                            c_in, ch3, ho, wo, wop, w2,
                            mask_bot, mask_right, n_per):
    for n in range(n_per):
        _one_image(x_ref, w_ref, y_ref, pool_ref, stats_ref, n,
                   c_in=c_in, ch3=ch3, ho=ho, wo=wo, wop=wop, w2=w2,
                   mask_bot=mask_bot, mask_right=mask_right)


def _one_image(x_ref, w_ref, y_ref, pool_ref, stats_ref, n, *,
               c_in, ch3, ho, wo, wop, w2, mask_bot, mask_right):
    xb = x_ref[n].astype(jnp.bfloat16)
    mrows = ho * wop
    c2 = 2 * c_in
    xe = xb[:, 0][0:ho]
    xo = xb[:, 1][0:ho]
    padr = wop - w2

    p1 = jnp.pad(xe, ((0, 0), (1, padr), (0, 0)))
    p2 = jnp.pad(xo, ((0, 0), (1, padr), (0, 0)))
    p0 = jnp.pad(xo[0:ho - 1], ((1, 0), (1, padr), (0, 0)))

    y = jnp.zeros((mrows, ch3), jnp.float32)
    for kh, p in ((0, p0), (1, p1), (2, p2)):
        main = p[:, 1:1 + wop, :].reshape(mrows, c2)
        shif = p[:, 0:wop, :].reshape(mrows, c2)
        y = y + jnp.dot(main, w_ref[kh * 2 * c2:kh * 2 * c2 + c2],
                        preferred_element_type=jnp.float32)
        y = y + jnp.dot(shif, w_ref[kh * 2 * c2 + c2:(kh + 1) * 2 * c2],
                        preferred_element_type=jnp.float32)
    y_ref[n] = y.astype(jnp.bfloat16)

    owp_idx = lax.rem(lax.broadcasted_iota(jnp.int32, (mrows, 1), 0),
                      jnp.int32(wop))
    ys = jnp.where(owp_idx < wo, y, 0.0)
    stats_ref[n, 0:1, :] = jnp.sum(ys, axis=0, keepdims=True)
    stats_ref[n, 1:2, :] = jnp.sum(ys * ys, axis=0, keepdims=True)

    neg = jnp.bfloat16(-1e30)
    shp = (ho, wop + 1, c2)
    d0 = lax.broadcasted_iota(jnp.int32, shp, 0)
    p0m = jnp.where(d0 == 0, neg, p0)
    p2m = jnp.where(d0 == ho - 1, neg, p2) if mask_bot else p2
    m1 = jnp.maximum(jnp.maximum(p0m, p1), p2m)
    d1 = lax.broadcasted_iota(jnp.int32, shp, 1)
    dl = lax.broadcasted_iota(jnp.int32, shp, 2)
    m1 = jnp.where((d1 == 0) & (dl >= c_in), neg, m1)
    if mask_right:
        m1 = jnp.where((d1 == wo) & (dl >= c_in), neg, m1)
    pa = m1[:, 1:1 + wop, :].reshape(mrows, c2)
    pb = m1[:, 0:wop, :].reshape(mrows, c2)
    pool = jnp.maximum(jnp.maximum(pa[:, 0:c_in], pa[:, c_in:c2]),
                       pb[:, c_in:c2])
    pool_ref[n] = pool


def _apply_kernel(y_ref, pool_ref, stats_ref, out_ref, *, ch3, c_in, m_total, eps):
    ssum = jnp.sum(stats_ref[:, 0, :], axis=0, keepdims=True)
    ssq = jnp.sum(stats_ref[:, 1, :], axis=0, keepdims=True)
    mean = ssum / m_total
    var = jnp.maximum(ssq / m_total - mean * mean, 0.0)
    inv_std = lax.rsqrt(var + eps)
    shift = -mean * inv_std
    y = y_ref[...].astype(jnp.float32)
    out_ref[:, 0:ch3] = jnp.maximum(y * inv_std + shift, 0.0)
    out_ref[:, ch3:ch3 + c_in] = pool_ref[...].astype(jnp.float32)


def kernel(x_nchw, w_oihw, bias, *, eps=1e-5):
    del bias

    N, C_in, H, W = x_nchw.shape
    Ch3 = w_oihw.shape[0]
    Ho = (H + 2 - 3) // 2 + 1
    Wo = (W + 2 - 3) // 2 + 1
    Wop = _ceil_to(Wo, 8)
    M = N * Ho * Wo
    Mr = Ho * Wop
    Cout = Ch3 + C_in

    cparams = pltpu.CompilerParams(dimension_semantics=("parallel",),
                                   vmem_limit_bytes=48 * 1024 * 1024)
    x = jnp.transpose(x_nchw, (0, 2, 3, 1))
    He, We = H + H % 2, W + W % 2
    if He != H or We != W:
        x = jnp.pad(x, ((0, 0), (0, He - H), (0, We - W), (0, 0)))
    xpp = x.reshape(N, He // 2, 2, We // 2, 2 * C_in)
    W2 = We // 2

    wt = jnp.transpose(w_oihw, (2, 3, 1, 0)).astype(jnp.bfloat16)
    blocks = []
    for kh in range(3):
        blocks.append(wt[kh, 1])
        blocks.append(wt[kh, 2])
        blocks.append(jnp.zeros((C_in, Ch3), jnp.bfloat16))
        blocks.append(wt[kh, 0])
    wcat = jnp.concatenate(blocks, axis=0)

    mask_bot = 2 * Ho - 1 >= H
    mask_right = 2 * Wo - 1 >= W
    NP = 4 if N % 4 == 0 else (2 if N % 2 == 0 else 1)

    cost1 = pl.CostEstimate(
        flops=2 * N * Mr * 6 * 2 * C_in * Ch3 + 12 * N * Mr * C_in,
        transcendentals=0,
        bytes_accessed=2 * N * (He * W2 * 2 * C_in + Mr * (Ch3 + C_in)) + 4 * N * 8 * Ch3)
    y_bf, pool_bf, stats = pl.pallas_call(
        functools.partial(_conv_pool_stats_kernel, c_in=C_in, ch3=Ch3,
                          ho=Ho, wo=Wo, wop=Wop, w2=W2,
                          mask_bot=mask_bot, mask_right=mask_right, n_per=NP),
        out_shape=(jax.ShapeDtypeStruct((N, Mr, Ch3), jnp.bfloat16),
                   jax.ShapeDtypeStruct((N, Mr, C_in), jnp.bfloat16),
                   jax.ShapeDtypeStruct((N, 8, Ch3), jnp.float32)),
        grid=(N // NP,),
        in_specs=[pl.BlockSpec((NP, He // 2, 2, W2, 2 * C_in),
                               lambda i: (i, 0, 0, 0, 0)),
                  pl.BlockSpec((12 * C_in, Ch3), lambda i: (0, 0))],
        out_specs=(pl.BlockSpec((NP, Mr, Ch3), lambda i: (i, 0, 0)),
                   pl.BlockSpec((NP, Mr, C_in), lambda i: (i, 0, 0)),
                   pl.BlockSpec((NP, 8, Ch3), lambda i: (i, 0, 0))),
        compiler_params=cparams,
        cost_estimate=cost1,
    )(xpp, wcat)

    M2 = N * Mr
    TM = min(4096, M2)
    n2 = _ceil_to(M2, TM) // TM
    cost2 = pl.CostEstimate(
        flops=3 * M2 * Ch3,
        transcendentals=0,
        bytes_accessed=2 * M2 * (Ch3 + C_in) + 4 * M2 * (Ch3 + C_in))
    fused = pl.pallas_call(
        functools.partial(_apply_kernel, ch3=Ch3, c_in=C_in, m_total=M, eps=eps),
        out_shape=jax.ShapeDtypeStruct((M2, Cout), jnp.float32),
        grid=(n2,),
        in_specs=[pl.BlockSpec((TM, Ch3), lambda i: (i, 0)),
                  pl.BlockSpec((TM, C_in), lambda i: (i, 0)),
                  pl.BlockSpec((N, 8, Ch3), lambda i: (0, 0, 0))],
        out_specs=pl.BlockSpec((TM, Cout), lambda i: (i, 0)),
        compiler_params=cparams,
        cost_estimate=cost2,
    )(y_bf.reshape(M2, Ch3), pool_bf.reshape(M2, C_in), stats)

    out = fused.reshape(N, Ho, Wop, Cout)[:, :, :Wo, :]
    return jnp.transpose(out, (0, 3, 1, 2))


# speedup vs baseline: 1.1984x; 1.0066x over previous
import functools

import jax
import jax.numpy as jnp
from jax import lax
from jax.experimental import pallas as pl
from jax.experimental.pallas import tpu as pltpu


def _ceil_to(x, m):
    return (x + m - 1) // m * m


def _conv_pool_stats_kernel(x_ref, w_ref, y_ref, pool_ref, stats_ref, *,
                            c_in, ch3, ho, wo, wop, w2,
                            mask_bot, mask_right, n_per):
    for n in range(n_per):
        _one_image(x_ref, w_ref, y_ref, pool_ref, stats_ref, n,
                   c_in=c_in, ch3=ch3, ho=ho, wo=wo, wop=wop, w2=w2,
                   mask_bot=mask_bot, mask_right=mask_right)


def _one_image(x_ref, w_ref, y_ref, pool_ref, stats_ref, n, *,
               c_in, ch3, ho, wo, wop, w2, mask_bot, mask_right):
    xb = x_ref[n].astype(jnp.bfloat16)
    mrows = ho * wop
    c2 = 2 * c_in
    xe = xb[:, 0][0:ho]
    xo = xb[:, 1][0:ho]
    padr = wop - w2

    p1 = jnp.pad(xe, ((0, 0), (1, padr), (0, 0)))
    p2 = jnp.pad(xo, ((0, 0), (1, padr), (0, 0)))
    p0 = jnp.pad(xo[0:ho - 1], ((1, 0), (1, padr), (0, 0)))

    y = jnp.zeros((mrows, ch3), jnp.float32)
    for kh, p in ((0, p0), (1, p1), (2, p2)):
        main = p[:, 1:1 + wop, :].reshape(mrows, c2)
        shif = p[:, 0:wop, :].reshape(mrows, c2)
        y = y + jnp.dot(main, w_ref[kh * 2 * c2:kh * 2 * c2 + c2],
                        preferred_element_type=jnp.float32)
        y = y + jnp.dot(shif, w_ref[kh * 2 * c2 + c2:(kh + 1) * 2 * c2],
                        preferred_element_type=jnp.float32)
    y_ref[n] = y.astype(jnp.bfloat16)

    owp_idx = lax.rem(lax.broadcasted_iota(jnp.int32, (mrows, 1), 0),
                      jnp.int32(wop))
    ys = jnp.where(owp_idx < wo, y, 0.0)
    stats_ref[n, 0:1, :] = jnp.sum(ys, axis=0, keepdims=True)
    stats_ref[n, 1:2, :] = jnp.sum(ys * ys, axis=0, keepdims=True)

    neg = jnp.bfloat16(-1e30)
    shp = (ho, wop + 1, c2)
    d0 = lax.broadcasted_iota(jnp.int32, shp, 0)
    p0m = jnp.where(d0 == 0, neg, p0)
    p2m = jnp.where(d0 == ho - 1, neg, p2) if mask_bot else p2
    m1 = jnp.maximum(jnp.maximum(p0m, p1), p2m)
    d1 = lax.broadcasted_iota(jnp.int32, shp, 1)
    dl = lax.broadcasted_iota(jnp.int32, shp, 2)
    m1 = jnp.where((d1 == 0) & (dl >= c_in), neg, m1)
    if mask_right:
        m1 = jnp.where((d1 == wo) & (dl >= c_in), neg, m1)
    pa = m1[:, 1:1 + wop, :].reshape(mrows, c2)
    pb = m1[:, 0:wop, :].reshape(mrows, c2)
    pool = jnp.maximum(jnp.maximum(pa[:, 0:c_in], pa[:, c_in:c2]),
                       pb[:, c_in:c2])
    pool_ref[n] = pool


def _apply_kernel(y_ref, pool_ref, stats_ref, out_ref, *, ch3, c_in, m_total, eps):
    ssum = jnp.sum(stats_ref[:, 0, :], axis=0, keepdims=True)
    ssq = jnp.sum(stats_ref[:, 1, :], axis=0, keepdims=True)
    mean = ssum / m_total
    var = jnp.maximum(ssq / m_total - mean * mean, 0.0)
    inv_std = lax.rsqrt(var + eps)
    shift = -mean * inv_std
    y = y_ref[...].astype(jnp.float32)
    out_ref[:, 0:ch3] = jnp.maximum(y * inv_std + shift, 0.0)
    out_ref[:, ch3:ch3 + c_in] = pool_ref[...].astype(jnp.float32)


def kernel(x_nchw, w_oihw, bias, *, eps=1e-5):
    del bias

    N, C_in, H, W = x_nchw.shape
    Ch3 = w_oihw.shape[0]
    Ho = (H + 2 - 3) // 2 + 1
    Wo = (W + 2 - 3) // 2 + 1
    Wop = _ceil_to(Wo, 8)
    M = N * Ho * Wo
    Mr = Ho * Wop
    Cout = Ch3 + C_in

    cparams = pltpu.CompilerParams(dimension_semantics=("parallel",),
                                   vmem_limit_bytes=48 * 1024 * 1024)
    x = jnp.transpose(x_nchw, (0, 2, 3, 1))
    He, We = H + H % 2, W + W % 2
    if He != H or We != W:
        x = jnp.pad(x, ((0, 0), (0, He - H), (0, We - W), (0, 0)))
    xpp = x.reshape(N, He // 2, 2, We // 2, 2 * C_in)
    W2 = We // 2

    wt = jnp.transpose(w_oihw, (2, 3, 1, 0)).astype(jnp.bfloat16)
    blocks = []
    for kh in range(3):
        blocks.append(wt[kh, 1])
        blocks.append(wt[kh, 2])
        blocks.append(jnp.zeros((C_in, Ch3), jnp.bfloat16))
        blocks.append(wt[kh, 0])
    wcat = jnp.concatenate(blocks, axis=0)

    mask_bot = 2 * Ho - 1 >= H
    mask_right = 2 * Wo - 1 >= W
    NP = 4 if N % 4 == 0 else (2 if N % 2 == 0 else 1)

    cost1 = pl.CostEstimate(
        flops=2 * N * Mr * 6 * 2 * C_in * Ch3 + 12 * N * Mr * C_in,
        transcendentals=0,
        bytes_accessed=2 * N * (He * W2 * 2 * C_in + Mr * (Ch3 + C_in)) + 4 * N * 8 * Ch3)
    y_bf, pool_bf, stats = pl.pallas_call(
        functools.partial(_conv_pool_stats_kernel, c_in=C_in, ch3=Ch3,
                          ho=Ho, wo=Wo, wop=Wop, w2=W2,
                          mask_bot=mask_bot, mask_right=mask_right, n_per=NP),
        out_shape=(jax.ShapeDtypeStruct((N, Mr, Ch3), jnp.bfloat16),
                   jax.ShapeDtypeStruct((N, Mr, C_in), jnp.bfloat16),
                   jax.ShapeDtypeStruct((N, 8, Ch3), jnp.float32)),
        grid=(N // NP,),
        in_specs=[pl.BlockSpec((NP, He // 2, 2, W2, 2 * C_in),
                               lambda i: (i, 0, 0, 0, 0)),
                  pl.BlockSpec((12 * C_in, Ch3), lambda i: (0, 0))],
        out_specs=(pl.BlockSpec((NP, Mr, Ch3), lambda i: (i, 0, 0)),
                   pl.BlockSpec((NP, Mr, C_in), lambda i: (i, 0, 0)),
                   pl.BlockSpec((NP, 8, Ch3), lambda i: (i, 0, 0))),
        compiler_params=cparams,
        cost_estimate=cost1,
    )(xpp, wcat)

    M2 = N * Mr
    TM = min(7168, M2)
    n2 = _ceil_to(M2, TM) // TM
    cost2 = pl.CostEstimate(
        flops=3 * M2 * Ch3,
        transcendentals=0,
        bytes_accessed=2 * M2 * (Ch3 + C_in) + 4 * M2 * (Ch3 + C_in))
    fused = pl.pallas_call(
        functools.partial(_apply_kernel, ch3=Ch3, c_in=C_in, m_total=M, eps=eps),
        out_shape=jax.ShapeDtypeStruct((M2, Cout), jnp.float32),
        grid=(n2,),
        in_specs=[pl.BlockSpec((TM, Ch3), lambda i: (i, 0)),
                  pl.BlockSpec((TM, C_in), lambda i: (i, 0)),
                  pl.BlockSpec((N, 8, Ch3), lambda i: (0, 0, 0))],
        out_specs=pl.BlockSpec((TM, Cout), lambda i: (i, 0)),
        compiler_params=cparams,
        cost_estimate=cost2,
    )(y_bf.reshape(M2, Ch3), pool_bf.reshape(M2, C_in), stats)

    out = fused.reshape(N, Ho, Wop, Cout)[:, :, :Wo, :]
    return jnp.transpose(out, (0, 3, 1, 2))


# speedup vs baseline: 1.2010x; 1.0021x over previous
import functools

import jax
import jax.numpy as jnp
from jax import lax
from jax.experimental import pallas as pl
from jax.experimental.pallas import tpu as pltpu


def _ceil_to(x, m):
    return (x + m - 1) // m * m


def _conv_pool_stats_kernel(x_ref, w_ref, y_ref, pool_ref, stats_ref, *,
                            c_in, ch3, ho, wo, wop, w2,
                            mask_bot, mask_right, n_per):
    for n in range(n_per):
        _one_image(x_ref, w_ref, y_ref, pool_ref, stats_ref, n,
                   c_in=c_in, ch3=ch3, ho=ho, wo=wo, wop=wop, w2=w2,
                   mask_bot=mask_bot, mask_right=mask_right)


def _one_image(x_ref, w_ref, y_ref, pool_ref, stats_ref, n, *,
               c_in, ch3, ho, wo, wop, w2, mask_bot, mask_right):
    xb = x_ref[n].astype(jnp.bfloat16)
    mrows = ho * wop
    c2 = 2 * c_in
    xe = xb[:, 0][0:ho]
    xo = xb[:, 1][0:ho]
    padr = wop - w2

    p1 = jnp.pad(xe, ((0, 0), (1, padr), (0, 0)))
    p2 = jnp.pad(xo, ((0, 0), (1, padr), (0, 0)))
    p0 = jnp.pad(xo[0:ho - 1], ((1, 0), (1, padr), (0, 0)))

    y = jnp.zeros((mrows, ch3), jnp.float32)
    for kh, p in ((0, p0), (1, p1), (2, p2)):
        main = p[:, 1:1 + wop, :].reshape(mrows, c2)
        shif = p[:, 0:wop, :].reshape(mrows, c2)
        y = y + jnp.dot(main, w_ref[kh * 2 * c2:kh * 2 * c2 + c2],
                        preferred_element_type=jnp.float32)
        y = y + jnp.dot(shif, w_ref[kh * 2 * c2 + c2:(kh + 1) * 2 * c2],
                        preferred_element_type=jnp.float32)
    y_ref[n] = y.astype(jnp.bfloat16)

    owp_idx = lax.rem(lax.broadcasted_iota(jnp.int32, (mrows, 1), 0),
                      jnp.int32(wop))
    ys = jnp.where(owp_idx < wo, y, 0.0)
    stats_ref[n, 0:1, :] = jnp.sum(ys, axis=0, keepdims=True)
    stats_ref[n, 1:2, :] = jnp.sum(ys * ys, axis=0, keepdims=True)

    neg = jnp.bfloat16(-1e30)
    shp = (ho, wop + 1, c2)
    d0 = lax.broadcasted_iota(jnp.int32, shp, 0)
    p0m = jnp.where(d0 == 0, neg, p0)
    p2m = jnp.where(d0 == ho - 1, neg, p2) if mask_bot else p2
    m1 = jnp.maximum(jnp.maximum(p0m, p1), p2m)
    d1 = lax.broadcasted_iota(jnp.int32, shp, 1)
    dl = lax.broadcasted_iota(jnp.int32, shp, 2)
    m1 = jnp.where((d1 == 0) & (dl >= c_in), neg, m1)
    if mask_right:
        m1 = jnp.where((d1 == wo) & (dl >= c_in), neg, m1)
    pa = m1[:, 1:1 + wop, :].reshape(mrows, c2)
    pb = m1[:, 0:wop, :].reshape(mrows, c2)
    pool = jnp.maximum(jnp.maximum(pa[:, 0:c_in], pa[:, c_in:c2]),
                       pb[:, c_in:c2])
    pool_ref[n] = pool


def _apply_kernel(y_ref, pool_ref, stats_ref, out_ref, *, ch3, c_in, m_total, eps):
    ssum = jnp.sum(stats_ref[:, 0, :], axis=0, keepdims=True)
    ssq = jnp.sum(stats_ref[:, 1, :], axis=0, keepdims=True)
    mean = ssum / m_total
    var = jnp.maximum(ssq / m_total - mean * mean, 0.0)
    inv_std = lax.rsqrt(var + eps)
    shift = -mean * inv_std
    y = y_ref[...].astype(jnp.float32)
    out_ref[:, 0:ch3] = jnp.maximum(y * inv_std + shift, 0.0)
    out_ref[:, ch3:ch3 + c_in] = pool_ref[...].astype(jnp.float32)


def kernel(x_nchw, w_oihw, bias, *, eps=1e-5):
    del bias

    N, C_in, H, W = x_nchw.shape
    Ch3 = w_oihw.shape[0]
    Ho = (H + 2 - 3) // 2 + 1
    Wo = (W + 2 - 3) // 2 + 1
    Wop = _ceil_to(Wo, 8)
    M = N * Ho * Wo
    Mr = Ho * Wop
    Cout = Ch3 + C_in

    cparams = pltpu.CompilerParams(dimension_semantics=("parallel",),
                                   vmem_limit_bytes=48 * 1024 * 1024)
    x = jnp.transpose(x_nchw, (0, 2, 3, 1))
    He, We = H + H % 2, W + W % 2
    if He != H or We != W:
        x = jnp.pad(x, ((0, 0), (0, He - H), (0, We - W), (0, 0)))
    xpp = x.reshape(N, He // 2, 2, We // 2, 2 * C_in)
    W2 = We // 2

    wt = jnp.transpose(w_oihw, (2, 3, 1, 0)).astype(jnp.bfloat16)
    blocks = []
    for kh in range(3):
        blocks.append(wt[kh, 1])
        blocks.append(wt[kh, 2])
        blocks.append(jnp.zeros((C_in, Ch3), jnp.bfloat16))
        blocks.append(wt[kh, 0])
    wcat = jnp.concatenate(blocks, axis=0)

    mask_bot = 2 * Ho - 1 >= H
    mask_right = 2 * Wo - 1 >= W
    NP = 8 if N % 8 == 0 else (2 if N % 2 == 0 else 1)

    cost1 = pl.CostEstimate(
        flops=2 * N * Mr * 6 * 2 * C_in * Ch3 + 12 * N * Mr * C_in,
        transcendentals=0,
        bytes_accessed=2 * N * (He * W2 * 2 * C_in + Mr * (Ch3 + C_in)) + 4 * N * 8 * Ch3)
    y_bf, pool_bf, stats = pl.pallas_call(
        functools.partial(_conv_pool_stats_kernel, c_in=C_in, ch3=Ch3,
                          ho=Ho, wo=Wo, wop=Wop, w2=W2,
                          mask_bot=mask_bot, mask_right=mask_right, n_per=NP),
        out_shape=(jax.ShapeDtypeStruct((N, Mr, Ch3), jnp.bfloat16),
                   jax.ShapeDtypeStruct((N, Mr, C_in), jnp.bfloat16),
                   jax.ShapeDtypeStruct((N, 8, Ch3), jnp.float32)),
        grid=(N // NP,),
        in_specs=[pl.BlockSpec((NP, He // 2, 2, W2, 2 * C_in),
                               lambda i: (i, 0, 0, 0, 0)),
                  pl.BlockSpec((12 * C_in, Ch3), lambda i: (0, 0))],
        out_specs=(pl.BlockSpec((NP, Mr, Ch3), lambda i: (i, 0, 0)),
                   pl.BlockSpec((NP, Mr, C_in), lambda i: (i, 0, 0)),
                   pl.BlockSpec((NP, 8, Ch3), lambda i: (i, 0, 0))),
        compiler_params=cparams,
        cost_estimate=cost1,
    )(xpp, wcat)

    M2 = N * Mr
    TM = min(7168, M2)
    n2 = _ceil_to(M2, TM) // TM
    cost2 = pl.CostEstimate(
        flops=3 * M2 * Ch3,
        transcendentals=0,
        bytes_accessed=2 * M2 * (Ch3 + C_in) + 4 * M2 * (Ch3 + C_in))
    fused = pl.pallas_call(
        functools.partial(_apply_kernel, ch3=Ch3, c_in=C_in, m_total=M, eps=eps),
        out_shape=jax.ShapeDtypeStruct((M2, Cout), jnp.float32),
        grid=(n2,),
        in_specs=[pl.BlockSpec((TM, Ch3), lambda i: (i, 0)),
                  pl.BlockSpec((TM, C_in), lambda i: (i, 0)),
                  pl.BlockSpec((N, 8, Ch3), lambda i: (0, 0, 0))],
        out_specs=pl.BlockSpec((TM, Cout), lambda i: (i, 0)),
        compiler_params=cparams,
        cost_estimate=cost2,
    )(y_bf.reshape(M2, Ch3), pool_bf.reshape(M2, C_in), stats)

    out = fused.reshape(N, Ho, Wop, Cout)[:, :, :Wo, :]
    return jnp.transpose(out, (0, 3, 1, 2))
